# baseline (speedup 1.0000x reference)
import jax
import jax.numpy as jnp
from jax import lax
from jax.experimental import pallas as pl
from jax.experimental.pallas import tpu as pltpu

_N = 5000
_P = 5120
_BLK = 128
_G = _P // _BLK
_M = 16
_IOU_T = 0.3
_SCORE_T = 0.2


def _row_feats(br):
    cx, cy = br[:, 0:1], br[:, 1:2]
    dx, dy = br[:, 3:4], br[:, 4:5]
    return (cx - dx * 0.5, cx + dx * 0.5, cy - dy * 0.5, cy + dy * 0.5,
            dx * dy, br[:, 7:8])


def _col_feats(bt):
    cx, cy = bt[0:1, :], bt[1:2, :]
    dx, dy = bt[3:4, :], bt[4:5, :]
    return (cx - dx * 0.5, cx + dx * 0.5, cy - dy * 0.5, cy + dy * 0.5,
            dx * dy, bt[7:8, :])


def _overlap(rowf, colf):
    x1r, x2r, y1r, y2r, ar, lr = rowf
    x1c, x2c, y1c, y2c, ac, lc = colf
    ix = jnp.maximum(0.0, jnp.minimum(x2r, x2c) - jnp.maximum(x1r, x1c))
    iy = jnp.maximum(0.0, jnp.minimum(y2r, y2c) - jnp.maximum(y1r, y1c))
    inter = ix * iy
    iou = inter / jnp.maximum(ar + ac - inter, 1e-6)
    same = lr == lc
    return iou, (iou > _IOU_T) & same


def _fused_kernel(br_ref, btb_ref, bt_ref, st_ref, bg_ref,
                  w1_ref, b1_ref, w2_ref, b2_ref,
                  supp_ref, out_ref, m_ref, g_ref):
    i = pl.program_id(0)

    @pl.when(i == 0)
    def _init():
        supp_ref[...] = (st_ref[...] <= _SCORE_T).astype(jnp.float32)

    rowf = _row_feats(br_ref[...])
    _, ovl_b = _overlap(rowf, _col_feats(btb_ref[...]))
    col_iota = lax.broadcasted_iota(jnp.int32, (1, _P), 1)
    row_iota = lax.broadcasted_iota(jnp.int32, (_BLK, 1), 0)
    lane2 = lax.broadcasted_iota(jnp.int32, (_BLK, _BLK), 1)
    row2 = lax.broadcasted_iota(jnp.int32, (_BLK, _BLK), 0)

    supp_old = supp_ref[...]
    base = pl.multiple_of(i * _BLK, _BLK)
    sub0 = supp_ref[0:1, pl.ds(base, _BLK)].T
    valid_col = sub0 == 0.0

    m_low = ovl_b.astype(jnp.float32) * (lane2 < row2).astype(jnp.float32)

    def fp_cond(c):
        k_row, prev = c
        return jnp.any(k_row != prev)

    def fp_body(c):
        k_row, _ = c
        s_col = jnp.sum(m_low * k_row, axis=1, keepdims=True)
        new_col = (valid_col & (s_col == 0.0)).astype(jnp.float32)
        return new_col.T, k_row

    k_row0 = valid_col.astype(jnp.float32).T
    k_row, _ = lax.while_loop(fp_cond, fp_body, (k_row0, k_row0 - 2.0))
    kept = k_row.T
    sub = 1.0 - kept

    iou, ov = _overlap(rowf, _col_feats(bt_ref[...]))
    ovf = ov.astype(jnp.float32)
    prop = (jnp.sum(ovf * kept, axis=0, keepdims=True) > 0).astype(jnp.float32)
    later = (col_iota >= (i + 1) * _BLK).astype(jnp.float32)
    supp_ref[...] = jnp.maximum(supp_old, prop * later)
    supp_ref[0:1, pl.ds(base, _BLK)] = sub.T

    m_ref[...] = jnp.where(ov, iou, -1.0)
    g_ref[...] = jnp.zeros((_M * _BLK, 8), jnp.float32)

    cnt = jnp.sum(ovf, axis=1, keepdims=True)
    trip = jnp.minimum(jnp.int32(_M), jnp.max(cnt).astype(jnp.int32))
    ci = col_iota
    bg = bg_ref[...]

    def body(k, carry):
        vp, ip = carry
        masked = m_ref[...]
        avail = (masked < vp) | ((masked == vp) & (ci > ip))
        mm = jnp.where(avail, masked, -2.0)
        v = jnp.max(mm, axis=1, keepdims=True)
        eq = mm == v
        idx = jnp.min(jnp.where(eq, ci, jnp.int32(2 ** 30)),
                      axis=1, keepdims=True)
        one = eq & (ci == idx)
        oh = (one & (v > _IOU_T)).astype(jnp.float32)
        g = jnp.dot(oh, bg, preferred_element_type=jnp.float32)
        g_ref[pl.ds(k * _BLK, _BLK), :] = jnp.concatenate(
            [g[:, 0:7], v], axis=1)
        return v, idx

    v0 = jnp.full((_BLK, 1), 2.0, jnp.float32)
    i0 = jnp.full((_BLK, 1), -1, jnp.int32)
    lax.fori_loop(0, trip, body, (v0, i0))

    gall = g_ref[...]
    feat = jnp.tanh(jnp.dot(gall, w1_ref[...],
                            preferred_element_type=jnp.float32) + b1_ref[...])
    lg_all = jnp.sum(feat * w2_ref[...], axis=1, keepdims=True) + b2_ref[...]

    groups, logits = [], []
    for k in range(_M):
        gk = gall[k * _BLK:(k + 1) * _BLK, :]
        valid = gk[:, 7:8] > _IOU_T
        groups.append(jnp.where(valid, gk, 0.0))
        logits.append(jnp.where(valid, lg_all[k * _BLK:(k + 1) * _BLK, :],
                                -1e9))

    lg = jnp.concatenate(logits, axis=1)
    m = jnp.max(lg, axis=1, keepdims=True)
    e = jnp.exp(lg - m)
    w = e / jnp.sum(e, axis=1, keepdims=True)

    merged = groups[0] * w[:, 0:1]
    for k in range(1, _M):
        merged = merged + groups[k] * w[:, k:k + 1]

    sizes = jnp.where(merged[:, 3:6] <= 0.0, br_ref[:, 3:6], merged[:, 3:6])
    out8 = jnp.concatenate([merged[:, 0:3], sizes, merged[:, 6:8]], axis=1)
    out_ref[...] = jnp.where(kept > 0.0, out8, 0.0)


def kernel(boxes, scores, labels, W1, b1, W2, b2):
    order = jnp.argsort(-scores)
    b = boxes[order]
    s = scores[order]
    l = labels[order].astype(jnp.float32)

    pad = _P - _N
    b7 = jnp.pad(b, ((0, pad), (0, 0)))
    lp = jnp.pad(l, (0, pad), constant_values=-1.0)
    sp = jnp.pad(s, (0, pad), constant_values=-1.0)

    br_all = jnp.concatenate([b7, lp[:, None]], axis=1)
    bt = br_all.T
    bg = jnp.concatenate([b7, jnp.zeros((_P, 1), jnp.float32)], axis=1)
    st = sp[None, :]

    w1p = jnp.pad(W1, ((0, 1), (0, 0)))
    b1r = b1[None, :]
    w2r = W2[:, 0][None, :]
    b2s = b2.reshape(1, 1)

    _, out = pl.pallas_call(
        _fused_kernel,
        grid=(_G,),
        in_specs=[
            pl.BlockSpec((_BLK, 8), lambda i: (i, 0)),
            pl.BlockSpec((8, _BLK), lambda i: (0, i)),
            pl.BlockSpec((8, _P), lambda i: (0, 0)),
            pl.BlockSpec((1, _P), lambda i: (0, 0)),
            pl.BlockSpec((_P, 8), lambda i: (0, 0)),
            pl.BlockSpec((8, 32), lambda i: (0, 0)),
            pl.BlockSpec((1, 32), lambda i: (0, 0)),
            pl.BlockSpec((1, 32), lambda i: (0, 0)),
            pl.BlockSpec((1, 1), lambda i: (0, 0)),
        ],
        out_specs=[
            pl.BlockSpec((1, _P), lambda i: (0, 0)),
            pl.BlockSpec((_BLK, 8), lambda i: (i, 0)),
        ],
        out_shape=[
            jax.ShapeDtypeStruct((1, _P), jnp.float32),
            jax.ShapeDtypeStruct((_P, 8), jnp.float32),
        ],
        scratch_shapes=[
            pltpu.VMEM((_BLK, _P), jnp.float32),
            pltpu.VMEM((_M * _BLK, 8), jnp.float32),
        ],
    )(br_all, bt, bt, st, bg, w1p, b1r, w2r, b2s)

    return out[:_N, :7]


# speedup vs baseline: 1.2097x; 1.2097x over previous
import jax
import jax.numpy as jnp
from jax import lax
from jax.experimental import pallas as pl
from jax.experimental.pallas import tpu as pltpu

_N = 5000
_P = 5120
_BLK = 128
_G = _P // _BLK
_M = 16
_IOU_T = 0.3
_SCORE_T = 0.2


def _row_feats(br):
    cx, cy = br[:, 0:1], br[:, 1:2]
    dx, dy = br[:, 3:4], br[:, 4:5]
    return (cx - dx * 0.5, cx + dx * 0.5, cy - dy * 0.5, cy + dy * 0.5,
            dx * dy, br[:, 7:8])


def _col_feats(bt):
    cx, cy = bt[0:1, :], bt[1:2, :]
    dx, dy = bt[3:4, :], bt[4:5, :]
    return (cx - dx * 0.5, cx + dx * 0.5, cy - dy * 0.5, cy + dy * 0.5,
            dx * dy, bt[7:8, :])


def _overlap(rowf, colf):
    x1r, x2r, y1r, y2r, ar, lr = rowf
    x1c, x2c, y1c, y2c, ac, lc = colf
    ix = jnp.maximum(0.0, jnp.minimum(x2r, x2c) - jnp.maximum(x1r, x1c))
    iy = jnp.maximum(0.0, jnp.minimum(y2r, y2c) - jnp.maximum(y1r, y1c))
    inter = ix * iy
    iou = inter / jnp.maximum(ar + ac - inter, 1e-6)
    same = lr == lc
    return iou, (iou > _IOU_T) & same


def _fused_kernel(br_ref, btb_ref, bt_ref, st_ref, bg_ref,
                  w1_ref, b1_ref, w2_ref, b2_ref,
                  supp_ref, out_ref, m_ref, g_ref):
    i = pl.program_id(0)

    @pl.when(i == 0)
    def _init():
        supp_ref[...] = (st_ref[...] <= _SCORE_T).astype(jnp.float32)

    rowf = _row_feats(br_ref[...])
    _, ovl_b = _overlap(rowf, _col_feats(btb_ref[...]))
    col_iota = lax.broadcasted_iota(jnp.int32, (1, _P), 1)
    row_iota = lax.broadcasted_iota(jnp.int32, (_BLK, 1), 0)
    lane2 = lax.broadcasted_iota(jnp.int32, (_BLK, _BLK), 1)
    row2 = lax.broadcasted_iota(jnp.int32, (_BLK, _BLK), 0)

    supp_old = supp_ref[...]
    base = pl.multiple_of(i * _BLK, _BLK)
    sub0 = supp_ref[0:1, pl.ds(base, _BLK)].T
    valid_col = sub0 == 0.0

    m_low = ovl_b.astype(jnp.float32) * (lane2 < row2).astype(jnp.float32)

    def fp_cond(c):
        k_row, prev = c
        return jnp.any(k_row != prev)

    def fp_body(c):
        k_row, _ = c
        s_col = jnp.sum(m_low * k_row, axis=1, keepdims=True)
        new_col = (valid_col & (s_col == 0.0)).astype(jnp.float32)
        return new_col.T, k_row

    k_row0 = valid_col.astype(jnp.float32).T
    k_row, _ = lax.while_loop(fp_cond, fp_body, (k_row0, k_row0 - 2.0))
    kept = k_row.T
    sub = 1.0 - kept

    iou, ov = _overlap(rowf, _col_feats(bt_ref[...]))
    ovf = ov.astype(jnp.float32)
    prop = (jnp.sum(ovf * kept, axis=0, keepdims=True) > 0).astype(jnp.float32)
    later = (col_iota >= (i + 1) * _BLK).astype(jnp.float32)
    supp_ref[...] = jnp.maximum(supp_old, prop * later)
    supp_ref[0:1, pl.ds(base, _BLK)] = sub.T

    m_ref[...] = jnp.where(ov, iou, -1.0)
    g_ref[...] = jnp.zeros((_M * _BLK, 8), jnp.float32)

    cnt = jnp.sum(ovf, axis=1, keepdims=True)
    trip = jnp.minimum(jnp.int32(_M), jnp.max(cnt).astype(jnp.int32))
    ci = col_iota
    bg = bg_ref[...]

    def body(k, _):
        masked = m_ref[...]
        v = jnp.max(masked, axis=1, keepdims=True)
        eq = masked == v
        idx = jnp.min(jnp.where(eq, ci, jnp.int32(2 ** 30)),
                      axis=1, keepdims=True)
        one = eq & (ci == idx)
        m_ref[...] = jnp.where(one, -2.0, masked)
        oh = (one & (v > _IOU_T)).astype(jnp.float32)
        g = jnp.dot(oh, bg, preferred_element_type=jnp.float32)
        g_ref[pl.ds(k * _BLK, _BLK), :] = jnp.concatenate(
            [g[:, 0:7], v], axis=1)
        return 0

    lax.fori_loop(0, trip, body, 0)

    gall = g_ref[...]
    feat = jnp.tanh(jnp.dot(gall, w1_ref[...],
                            preferred_element_type=jnp.float32) + b1_ref[...])
    lg_all = jnp.sum(feat * w2_ref[...], axis=1, keepdims=True) + b2_ref[...]

    groups, logits = [], []
    for k in range(_M):
        gk = gall[k * _BLK:(k + 1) * _BLK, :]
        valid = gk[:, 7:8] > _IOU_T
        groups.append(jnp.where(valid, gk, 0.0))
        logits.append(jnp.where(valid, lg_all[k * _BLK:(k + 1) * _BLK, :],
                                -1e9))

    lg = jnp.concatenate(logits, axis=1)
    m = jnp.max(lg, axis=1, keepdims=True)
    e = jnp.exp(lg - m)
    w = e / jnp.sum(e, axis=1, keepdims=True)

    merged = groups[0] * w[:, 0:1]
    for k in range(1, _M):
        merged = merged + groups[k] * w[:, k:k + 1]

    sizes = jnp.where(merged[:, 3:6] <= 0.0, br_ref[:, 3:6], merged[:, 3:6])
    out8 = jnp.concatenate([merged[:, 0:3], sizes, merged[:, 6:8]], axis=1)
    out_ref[...] = jnp.where(kept > 0.0, out8, 0.0)


def kernel(boxes, scores, labels, W1, b1, W2, b2):
    order = jnp.argsort(-scores)
    b = boxes[order]
    s = scores[order]
    l = labels[order].astype(jnp.float32)

    pad = _P - _N
    b7 = jnp.pad(b, ((0, pad), (0, 0)))
    lp = jnp.pad(l, (0, pad), constant_values=-1.0)
    sp = jnp.pad(s, (0, pad), constant_values=-1.0)

    br_all = jnp.concatenate([b7, lp[:, None]], axis=1)
    bt = br_all.T
    bg = jnp.concatenate([b7, jnp.zeros((_P, 1), jnp.float32)], axis=1)
    st = sp[None, :]

    w1p = jnp.pad(W1, ((0, 1), (0, 0)))
    b1r = b1[None, :]
    w2r = W2[:, 0][None, :]
    b2s = b2.reshape(1, 1)

    _, out = pl.pallas_call(
        _fused_kernel,
        grid=(_G,),
        in_specs=[
            pl.BlockSpec((_BLK, 8), lambda i: (i, 0)),
            pl.BlockSpec((8, _BLK), lambda i: (0, i)),
            pl.BlockSpec((8, _P), lambda i: (0, 0)),
            pl.BlockSpec((1, _P), lambda i: (0, 0)),
            pl.BlockSpec((_P, 8), lambda i: (0, 0)),
            pl.BlockSpec((8, 32), lambda i: (0, 0)),
            pl.BlockSpec((1, 32), lambda i: (0, 0)),
            pl.BlockSpec((1, 32), lambda i: (0, 0)),
            pl.BlockSpec((1, 1), lambda i: (0, 0)),
        ],
        out_specs=[
            pl.BlockSpec((1, _P), lambda i: (0, 0)),
            pl.BlockSpec((_BLK, 8), lambda i: (i, 0)),
        ],
        out_shape=[
            jax.ShapeDtypeStruct((1, _P), jnp.float32),
            jax.ShapeDtypeStruct((_P, 8), jnp.float32),
        ],
        scratch_shapes=[
            pltpu.VMEM((_BLK, _P), jnp.float32),
            pltpu.VMEM((_M * _BLK, 8), jnp.float32),
        ],
    )(br_all, bt, bt, st, bg, w1p, b1r, w2r, b2s)

    return out[:_N, :7]
